# baseline (speedup 1.0000x reference)
import functools

import jax
import jax.numpy as jnp
from jax.experimental import pallas as pl
from jax.experimental.pallas import tpu as pltpu

_LANES = 512
_GROUPS = _LANES // 4
_BLOCK_ROWS = 256


def _loss_kernel(xp_ref, xt_ref, sc_ref, out_ref, acc_ref, cnt_ref, *, m_rows, n_blocks):
    gi = pl.program_id(0)

    @pl.when(gi == 0)
    def _init():
        acc_ref[...] = jnp.zeros_like(acc_ref)
        cnt_ref[...] = jnp.zeros_like(cnt_ref)

    row0 = gi * _BLOCK_ROWS
    d = xp_ref[...] - xt_ref[...]
    ad = jnp.abs(d)
    pe = jnp.where(ad < 1.0, 0.5 * d * d, ad - 0.5)

    valid_e = (jax.lax.broadcasted_iota(jnp.int32, (_BLOCK_ROWS, _LANES), 0) + row0) < m_rows
    pe = jnp.where(valid_e, pe, 0.0)

    li = jax.lax.broadcasted_iota(jnp.int32, (_LANES, _GROUPS), 0)
    ji = jax.lax.broadcasted_iota(jnp.int32, (_LANES, _GROUPS), 1)
    sel = ((li // 4) == ji).astype(jnp.float32)
    rs = jax.lax.dot(pe, sel, preferred_element_type=jnp.float32)

    valid_r = (jax.lax.broadcasted_iota(jnp.int32, (_BLOCK_ROWS, _GROUPS), 0) + row0) < m_rows
    maskf = jnp.where(valid_r & (sc_ref[...] == 1), 1.0, 0.0)

    acc_ref[...] += jnp.sum(rs * maskf, axis=0, keepdims=True)
    cnt_ref[...] += jnp.sum(maskf, axis=0, keepdims=True)

    @pl.when(gi == n_blocks - 1)
    def _finish():
        total = jnp.sum(acc_ref[...])
        n_active = 4.0 * jnp.sum(cnt_ref[...])
        out_ref[...] = (total / n_active).reshape(1, 1)


@jax.jit
def _run(y_pred, y_true, score):
    n_elems = y_pred.size
    m_rows = n_elems // _LANES
    n_blocks = pl.cdiv(m_rows, _BLOCK_ROWS)

    xp = y_pred.reshape(m_rows, _LANES)
    xt = y_true.reshape(m_rows, _LANES)
    sc = score.reshape(m_rows, _GROUPS)

    out = pl.pallas_call(
        functools.partial(_loss_kernel, m_rows=m_rows, n_blocks=n_blocks),
        grid=(n_blocks,),
        in_specs=[
            pl.BlockSpec((_BLOCK_ROWS, _LANES), lambda i: (i, 0)),
            pl.BlockSpec((_BLOCK_ROWS, _LANES), lambda i: (i, 0)),
            pl.BlockSpec((_BLOCK_ROWS, _GROUPS), lambda i: (i, 0)),
        ],
        out_specs=pl.BlockSpec((1, 1), lambda i: (0, 0)),
        out_shape=jax.ShapeDtypeStruct((1, 1), jnp.float32),
        scratch_shapes=[
            pltpu.VMEM((1, _GROUPS), jnp.float32),
            pltpu.VMEM((1, _GROUPS), jnp.float32),
        ],
    )(xp, xt, sc)
    return out.reshape(())


def kernel(y_pred, y_true, y_true_score):
    return _run(y_pred, y_true, y_true_score.astype(jnp.int32))


# speedup vs baseline: 48.2129x; 48.2129x over previous
import functools

import jax
import jax.numpy as jnp
from jax.experimental import pallas as pl
from jax.experimental.pallas import tpu as pltpu

_BLOCK = 512


def _loss_kernel(xp_ref, xt_ref, sc_ref, out_ref, acc_ref, cnt_ref, *, m_rows, n_blocks):
    gi = pl.program_id(0)

    @pl.when(gi == 0)
    def _init():
        acc_ref[...] = jnp.zeros_like(acc_ref)
        cnt_ref[...] = jnp.zeros_like(cnt_ref)

    d = xp_ref[...] - xt_ref[...]
    ad = jnp.abs(d)
    mn = jnp.minimum(ad, 1.0)
    pe = mn * (ad - 0.5 * mn)

    row0 = gi * _BLOCK
    valid = (jax.lax.broadcasted_iota(jnp.int32, (_BLOCK, 2, 128), 0) + row0) < m_rows
    mb = (sc_ref[...] == 1) & valid

    masked = jnp.zeros((_BLOCK, 128), jnp.float32)
    for s in range(8):
        masked += jnp.where(mb[:, s // 4, :], pe[:, s, :], 0.0)

    acc_ref[...] += jnp.sum(masked).reshape(1, 1)
    cnt_ref[...] += jnp.sum(jnp.where(mb, 1.0, 0.0)).reshape(1, 1)

    @pl.when(gi == n_blocks - 1)
    def _finish():
        out_ref[...] = acc_ref[...] / (4.0 * cnt_ref[...])


@jax.jit
def _run(y_pred, y_true, score):
    n = y_pred.shape[0]
    m_rows = n // 256
    n_blocks = pl.cdiv(m_rows, _BLOCK)

    def as_dense(x):
        return x.reshape(m_rows, 2, 128, 4).transpose(0, 1, 3, 2).reshape(m_rows, 8, 128)

    xp = as_dense(y_pred)
    xt = as_dense(y_true)
    sc = score.reshape(m_rows, 2, 128)

    out = pl.pallas_call(
        functools.partial(_loss_kernel, m_rows=m_rows, n_blocks=n_blocks),
        grid=(n_blocks,),
        in_specs=[
            pl.BlockSpec((_BLOCK, 8, 128), lambda i: (i, 0, 0)),
            pl.BlockSpec((_BLOCK, 8, 128), lambda i: (i, 0, 0)),
            pl.BlockSpec((_BLOCK, 2, 128), lambda i: (i, 0, 0)),
        ],
        out_specs=pl.BlockSpec((1, 1), lambda i: (0, 0)),
        out_shape=jax.ShapeDtypeStruct((1, 1), jnp.float32),
        scratch_shapes=[
            pltpu.VMEM((1, 1), jnp.float32),
            pltpu.VMEM((1, 1), jnp.float32),
        ],
    )(xp, xt, sc)
    return out.reshape(())


def kernel(y_pred, y_true, y_true_score):
    return _run(y_pred, y_true, y_true_score.astype(jnp.int32))
